# baseline (speedup 1.0000x reference)
import functools

import jax
import jax.numpy as jnp
from jax import lax
from jax.experimental import pallas as pl
from jax.experimental.pallas import tpu as pltpu
from jax.experimental.pallas import tpu_sc as plsc

_BATCH = 16384

_NS = 16
_CHUNK = 128
_BW = _BATCH // _NS
_NCHUNK = _BW // _CHUNK


def _mf_body(x_hbm, y_hbm, ut_hbm, it_hbm, out_hbm,
             xv, yv, uidx, vidx, ubuf, vbuf, accv, allv, outv,
             usem, vsem):
    wid = lax.axis_index("s")
    rows_pw = _NCHUNK

    pltpu.sync_copy(x_hbm.at[pl.ds(wid * rows_pw, rows_pw), :], xv)
    pltpu.sync_copy(y_hbm.at[pl.ds(wid * rows_pw, rows_pw), :], yv)

    for c in range(_NCHUNK):
        for k in range(_CHUNK // 16):
            sl = pl.ds(16 * k, 16)
            xs = xv[c, sl] * 2
            uidx[2 * c, sl] = xs
            uidx[2 * c + 1, sl] = xs + 1
            ys = yv[c, sl] * 2
            vidx[2 * c, sl] = ys
            vidx[2 * c + 1, sl] = ys + 1

    acc = jnp.zeros((16,), jnp.float32)
    for j in range(2 * _NCHUNK):
        cu = pltpu.async_copy(ut_hbm.at[uidx.at[j]], ubuf, usem)
        cv = pltpu.async_copy(it_hbm.at[vidx.at[j]], vbuf, vsem)
        cu.wait()
        cv.wait()
        for k in range(_CHUNK // 16):
            sl = pl.ds(16 * k, 16)
            acc = acc + ubuf[sl] * vbuf[sl]

    accv[...] = acc
    pltpu.sync_copy(accv, out_hbm.at[pl.ds(wid * 16, 16)])
    plsc.subcore_barrier()

    @pl.when(wid == 0)
    def _():
        pltpu.sync_copy(out_hbm.at[pl.ds(0, _NS * 16)], allv)
        tot = jnp.zeros((16,), jnp.float32)
        for r in range(_NS):
            tot = tot + allv[pl.ds(16 * r, 16)]
        s = tot[0]
        for l in range(1, 16):
            s = s + tot[l]
        outv[...] = jnp.broadcast_to(s, (16,))
        pltpu.sync_copy(outv, out_hbm.at[pl.ds(_NS * 16, 16)])


@functools.partial(
    pl.kernel,
    mesh=plsc.VectorSubcoreMesh(core_axis_name="c", subcore_axis_name="s",
                                num_cores=1),
    out_type=jax.ShapeDtypeStruct(((_NS + 1) * 16,), jnp.float32),
    scratch_types=[
        pltpu.VMEM((_NCHUNK, _CHUNK), jnp.int32),
        pltpu.VMEM((_NCHUNK, _CHUNK), jnp.int32),
        pltpu.VMEM((2 * _NCHUNK, _CHUNK), jnp.int32),
        pltpu.VMEM((2 * _NCHUNK, _CHUNK), jnp.int32),
        pltpu.VMEM((_CHUNK,), jnp.float32),
        pltpu.VMEM((_CHUNK,), jnp.float32),
        pltpu.VMEM((16,), jnp.float32),
        pltpu.VMEM((_NS * 16,), jnp.float32),
        pltpu.VMEM((16,), jnp.float32),
        pltpu.SemaphoreType.DMA,
        pltpu.SemaphoreType.DMA,
    ],
)
def _mf(x_hbm, y_hbm, ut_hbm, it_hbm, out_hbm, *scratch):
    _mf_body(x_hbm, y_hbm, ut_hbm, it_hbm, out_hbm, *scratch)


def kernel(x, y, user_table, item_table):
    x2 = x.reshape(128, 128).astype(jnp.int32)
    y2 = y.reshape(128, 128).astype(jnp.int32)
    out = _mf(x2, y2, user_table.reshape(-1), item_table.reshape(-1))
    return out[_NS * 16]


# speedup vs baseline: 1.0046x; 1.0046x over previous
import functools

import jax
import jax.numpy as jnp
from jax import lax
from jax.experimental import pallas as pl
from jax.experimental.pallas import tpu as pltpu
from jax.experimental.pallas import tpu_sc as plsc

_BATCH = 16384

_NS = 16
_CHUNK = 128
_BW = _BATCH // _NS
_NCHUNK = _BW // _CHUNK


def _mf_body(x_hbm, y_hbm, ut_hbm, it_hbm, out_hbm,
             xv, yv, uidx, vidx, ubuf, vbuf, accv, allv, outv,
             usem, vsem):
    wid = lax.axis_index("s")
    rows_pw = _NCHUNK

    pltpu.sync_copy(x_hbm.at[pl.ds(wid * rows_pw, rows_pw), :], xv)
    pltpu.sync_copy(y_hbm.at[pl.ds(wid * rows_pw, rows_pw), :], yv)

    for c in range(_NCHUNK):
        for k in range(_CHUNK // 16):
            sl = pl.ds(16 * k, 16)
            xs = xv[c, sl] * 2
            uidx[2 * c, sl] = xs
            uidx[2 * c + 1, sl] = xs + 1
            ys = yv[c, sl] * 2
            vidx[2 * c, sl] = ys
            vidx[2 * c + 1, sl] = ys + 1

    copies = []
    for j in range(2 * _NCHUNK):
        copies.append(pltpu.async_copy(ut_hbm.at[uidx.at[j]], ubuf.at[j], usem))
        copies.append(pltpu.async_copy(it_hbm.at[vidx.at[j]], vbuf.at[j], vsem))
    for cp in copies:
        cp.wait()

    acc = jnp.zeros((16,), jnp.float32)
    for j in range(2 * _NCHUNK):
        for k in range(_CHUNK // 16):
            sl = pl.ds(16 * k, 16)
            acc = acc + ubuf[j, sl] * vbuf[j, sl]

    accv[...] = acc
    pltpu.sync_copy(accv, out_hbm.at[pl.ds(wid * 16, 16)])
    plsc.subcore_barrier()

    @pl.when(wid == 0)
    def _():
        pltpu.sync_copy(out_hbm.at[pl.ds(0, _NS * 16)], allv)
        tot = jnp.zeros((16,), jnp.float32)
        for r in range(_NS):
            tot = tot + allv[pl.ds(16 * r, 16)]
        s = tot[0]
        for l in range(1, 16):
            s = s + tot[l]
        outv[...] = jnp.broadcast_to(s, (16,))
        pltpu.sync_copy(outv, out_hbm.at[pl.ds(_NS * 16, 16)])


@functools.partial(
    pl.kernel,
    mesh=plsc.VectorSubcoreMesh(core_axis_name="c", subcore_axis_name="s",
                                num_cores=1),
    out_type=jax.ShapeDtypeStruct(((_NS + 1) * 16,), jnp.float32),
    scratch_types=[
        pltpu.VMEM((_NCHUNK, _CHUNK), jnp.int32),
        pltpu.VMEM((_NCHUNK, _CHUNK), jnp.int32),
        pltpu.VMEM((2 * _NCHUNK, _CHUNK), jnp.int32),
        pltpu.VMEM((2 * _NCHUNK, _CHUNK), jnp.int32),
        pltpu.VMEM((2 * _NCHUNK, _CHUNK), jnp.float32),
        pltpu.VMEM((2 * _NCHUNK, _CHUNK), jnp.float32),
        pltpu.VMEM((16,), jnp.float32),
        pltpu.VMEM((_NS * 16,), jnp.float32),
        pltpu.VMEM((16,), jnp.float32),
        pltpu.SemaphoreType.DMA,
        pltpu.SemaphoreType.DMA,
    ],
)
def _mf(x_hbm, y_hbm, ut_hbm, it_hbm, out_hbm, *scratch):
    _mf_body(x_hbm, y_hbm, ut_hbm, it_hbm, out_hbm, *scratch)


def kernel(x, y, user_table, item_table):
    x2 = x.reshape(128, 128).astype(jnp.int32)
    y2 = y.reshape(128, 128).astype(jnp.int32)
    out = _mf(x2, y2, user_table.reshape(-1), item_table.reshape(-1))
    return out[_NS * 16]


# speedup vs baseline: 1.0050x; 1.0004x over previous
import functools

import jax
import jax.numpy as jnp
from jax import lax
from jax.experimental import pallas as pl
from jax.experimental.pallas import tpu as pltpu
from jax.experimental.pallas import tpu_sc as plsc

_BATCH = 16384

_NS = 16
_CHUNK = 128
_BW = _BATCH // _NS
_NCHUNK = _BW // _CHUNK
_NROW = 2 * _NCHUNK


def _mf_body(x_hbm, y_hbm, ut_hbm, it_hbm, out_hbm,
             xv, yv, uidx, vidx, ubuf, vbuf, accv, allv, outv,
             usem, vsem):
    wid = lax.axis_index("s")

    pltpu.sync_copy(x_hbm.at[pl.ds(wid * _BW, _BW)], xv)
    pltpu.sync_copy(y_hbm.at[pl.ds(wid * _BW, _BW)], yv)

    for c in range(_NCHUNK):
        for k in range(_CHUNK // 16):
            sl = pl.ds(16 * k, 16)
            src = pl.ds(_CHUNK * c + 16 * k, 16)
            xs = xv[src] * 2
            uidx[2 * c, sl] = xs
            uidx[2 * c + 1, sl] = xs + 1
            ys = yv[src] * 2
            vidx[2 * c, sl] = ys
            vidx[2 * c + 1, sl] = ys + 1

    copies = []
    for j in range(_NROW):
        copies.append(pltpu.async_copy(ut_hbm.at[uidx.at[j]], ubuf.at[j], usem))
        copies.append(pltpu.async_copy(it_hbm.at[vidx.at[j]], vbuf.at[j], vsem))
    for cp in copies:
        cp.wait()

    acc = jnp.zeros((16,), jnp.float32)
    for j in range(_NROW):
        for k in range(_CHUNK // 16):
            sl = pl.ds(16 * k, 16)
            acc = acc + ubuf[j, sl] * vbuf[j, sl]

    accv[...] = acc
    pltpu.sync_copy(accv, out_hbm.at[pl.ds(wid * 16, 16)])
    plsc.subcore_barrier()

    @pl.when(wid == 0)
    def _():
        pltpu.sync_copy(out_hbm.at[pl.ds(0, _NS * 16)], allv)
        tot = jnp.zeros((16,), jnp.float32)
        for r in range(_NS):
            tot = tot + allv[pl.ds(16 * r, 16)]
        s = tot[0]
        for l in range(1, 16):
            s = s + tot[l]
        outv[...] = jnp.broadcast_to(s, (16,))
        pltpu.sync_copy(outv, out_hbm.at[pl.ds(_NS * 16, 16)])


@functools.partial(
    pl.kernel,
    mesh=plsc.VectorSubcoreMesh(core_axis_name="c", subcore_axis_name="s",
                                num_cores=1),
    out_type=jax.ShapeDtypeStruct(((_NS + 1) * 16,), jnp.float32),
    scratch_types=[
        pltpu.VMEM((_BW,), jnp.int32),
        pltpu.VMEM((_BW,), jnp.int32),
        pltpu.VMEM((_NROW, _CHUNK), jnp.int32),
        pltpu.VMEM((_NROW, _CHUNK), jnp.int32),
        pltpu.VMEM((_NROW, _CHUNK), jnp.float32),
        pltpu.VMEM((_NROW, _CHUNK), jnp.float32),
        pltpu.VMEM((16,), jnp.float32),
        pltpu.VMEM((_NS * 16,), jnp.float32),
        pltpu.VMEM((16,), jnp.float32),
        pltpu.SemaphoreType.DMA,
        pltpu.SemaphoreType.DMA,
    ],
)
def _mf(x_hbm, y_hbm, ut_hbm, it_hbm, out_hbm, *scratch):
    _mf_body(x_hbm, y_hbm, ut_hbm, it_hbm, out_hbm, *scratch)


def kernel(x, y, user_table, item_table):
    out = _mf(x, y, user_table.reshape(-1), item_table.reshape(-1))
    return out[_NS * 16]


# speedup vs baseline: 18.4291x; 18.3366x over previous
import functools

import jax
import jax.numpy as jnp
from jax import lax
from jax.experimental import pallas as pl
from jax.experimental.pallas import tpu as pltpu
from jax.experimental.pallas import tpu_sc as plsc

_BATCH = 16384

_NS = 16
_CHUNK = 128
_BW = _BATCH // _NS
_NCHUNK = _BW // _CHUNK


def _mf_body(x_hbm, y_hbm, u0_hbm, u1_hbm, i0_hbm, i1_hbm, out_hbm,
             xv, yv, u0b, u1b, i0b, i1b, accv, allv, outv,
             usem, vsem):
    wid = lax.axis_index("s")

    pltpu.sync_copy(x_hbm.at[pl.ds(wid * _BW, _BW)], xv)
    pltpu.sync_copy(y_hbm.at[pl.ds(wid * _BW, _BW)], yv)

    copies = []
    for c in range(_NCHUNK):
        xi = xv.at[pl.ds(c * _CHUNK, _CHUNK)]
        yi = yv.at[pl.ds(c * _CHUNK, _CHUNK)]
        copies.append(pltpu.async_copy(u0_hbm.at[xi], u0b.at[c], usem))
        copies.append(pltpu.async_copy(u1_hbm.at[xi], u1b.at[c], usem))
        copies.append(pltpu.async_copy(i0_hbm.at[yi], i0b.at[c], vsem))
        copies.append(pltpu.async_copy(i1_hbm.at[yi], i1b.at[c], vsem))
    for cp in copies:
        cp.wait()

    acc = jnp.zeros((16,), jnp.float32)
    for c in range(_NCHUNK):
        for k in range(_CHUNK // 16):
            sl = pl.ds(16 * k, 16)
            acc = acc + u0b[c, sl] * i0b[c, sl] + u1b[c, sl] * i1b[c, sl]

    accv[...] = acc
    pltpu.sync_copy(accv, out_hbm.at[pl.ds(wid * 16, 16)])
    plsc.subcore_barrier()

    @pl.when(wid == 0)
    def _():
        pltpu.sync_copy(out_hbm.at[pl.ds(0, _NS * 16)], allv)
        tot = jnp.zeros((16,), jnp.float32)
        for r in range(_NS):
            tot = tot + allv[pl.ds(16 * r, 16)]
        s = tot[0]
        for l in range(1, 16):
            s = s + tot[l]
        outv[...] = jnp.broadcast_to(s, (16,))
        pltpu.sync_copy(outv, out_hbm.at[pl.ds(_NS * 16, 16)])


@functools.partial(
    pl.kernel,
    mesh=plsc.VectorSubcoreMesh(core_axis_name="c", subcore_axis_name="s",
                                num_cores=1),
    out_type=jax.ShapeDtypeStruct(((_NS + 1) * 16,), jnp.float32),
    scratch_types=[
        pltpu.VMEM((_BW,), jnp.int32),
        pltpu.VMEM((_BW,), jnp.int32),
        pltpu.VMEM((_NCHUNK, _CHUNK), jnp.float32),
        pltpu.VMEM((_NCHUNK, _CHUNK), jnp.float32),
        pltpu.VMEM((_NCHUNK, _CHUNK), jnp.float32),
        pltpu.VMEM((_NCHUNK, _CHUNK), jnp.float32),
        pltpu.VMEM((16,), jnp.float32),
        pltpu.VMEM((_NS * 16,), jnp.float32),
        pltpu.VMEM((16,), jnp.float32),
        pltpu.SemaphoreType.DMA,
        pltpu.SemaphoreType.DMA,
    ],
)
def _mf(x_hbm, y_hbm, u0_hbm, u1_hbm, i0_hbm, i1_hbm, out_hbm, *scratch):
    _mf_body(x_hbm, y_hbm, u0_hbm, u1_hbm, i0_hbm, i1_hbm, out_hbm, *scratch)


def kernel(x, y, user_table, item_table):
    out = _mf(x, y,
              user_table[:, 0], user_table[:, 1],
              item_table[:, 0], item_table[:, 1])
    return out[_NS * 16]
